# baseline (speedup 1.0000x reference)
import functools

import jax
import jax.numpy as jnp
from jax import lax
from jax.experimental import pallas as pl
from jax.experimental.pallas import tpu as pltpu
from jax.experimental.pallas import tpu_sc as plsc

_N, _C, _HW = 4, 768, 1024


def _sc_partial_sums(x):
    info = plsc.get_sparse_core_info()
    nc, ns = info.num_cores, info.num_subcores
    nw = nc * ns
    cpb = nw // _N
    rpw = _HW // cpb
    mesh = plsc.VectorSubcoreMesh(core_axis_name="c", subcore_axis_name="s")

    @functools.partial(
        pl.kernel, mesh=mesh,
        out_type=jax.ShapeDtypeStruct((nw, _C), jnp.float32),
        scratch_types=[
            pltpu.VMEM((rpw, _C), jnp.float32),
            pltpu.VMEM((_C,), jnp.float32),
        ],
    )
    def k(x_hbm, out_hbm, buf_v, acc_v):
        wid = lax.axis_index("s") * nc + lax.axis_index("c")
        n = wid // cpb
        r = wid % cpb
        pltpu.sync_copy(x_hbm.at[n, pl.ds(r * rpw, rpw)], buf_v)
        for cc in range(_C // 16):
            sl = pl.ds(cc * 16, 16)

            def body(i, acc):
                return acc + buf_v[i, sl]

            acc_v[sl] = lax.fori_loop(0, rpw, body,
                                      jnp.zeros((16,), jnp.float32))
        pltpu.sync_copy(acc_v, out_hbm.at[wid])

    return k(x)


def _epilogue_kernel(p_ref, wt_ref, b_ref, o_ref):
    m = jnp.sum(p_ref[...], axis=0, keepdims=True) * (1.0 / _HW)
    pooled = jnp.dot(m, wt_ref[...],
                     preferred_element_type=jnp.float32) + b_ref[...]
    o_ref[0] = jnp.broadcast_to(pooled, o_ref.shape[1:])


def kernel(features, depth, W, b):
    del depth
    N, C, H, Wd = features.shape
    O = W.shape[0]
    HW = H * Wd
    x = features.transpose(0, 2, 3, 1).reshape(N, HW, C)
    partials = _sc_partial_sums(x)
    nw = partials.shape[0]
    cpb = nw // N
    wt = W.reshape(O, C).T
    b2 = b.reshape(1, O)
    out = pl.pallas_call(
        _epilogue_kernel,
        grid=(N,),
        in_specs=[
            pl.BlockSpec((cpb, C), lambda i: (i, 0)),
            pl.BlockSpec((C, O), lambda i: (0, 0)),
            pl.BlockSpec((1, O), lambda i: (0, 0)),
        ],
        out_specs=pl.BlockSpec((1, HW, O), lambda i: (i, 0, 0)),
        out_shape=jax.ShapeDtypeStruct((N, HW, O), jnp.float32),
    )(partials, wt, b2)
    return out.reshape(N, H, Wd, O).transpose(0, 3, 1, 2)


# speedup vs baseline: 1.4598x; 1.4598x over previous
import functools

import jax
import jax.numpy as jnp
from jax import lax
from jax.experimental import pallas as pl
from jax.experimental.pallas import tpu as pltpu
from jax.experimental.pallas import tpu_sc as plsc

_N, _C, _HW = 4, 768, 1024


def _sc_partial_sums(x):
    info = plsc.get_sparse_core_info()
    nc, ns = info.num_cores, info.num_subcores
    nw = nc * ns
    cpb = nw // _N
    rpw = _HW // cpb
    mesh = plsc.VectorSubcoreMesh(core_axis_name="c", subcore_axis_name="s")

    @functools.partial(
        pl.kernel, mesh=mesh,
        out_type=jax.ShapeDtypeStruct((nw, _C), jnp.float32),
        scratch_types=[
            pltpu.VMEM((rpw, _C), jnp.float32),
            pltpu.VMEM((_C,), jnp.float32),
        ],
    )
    def k(x_hbm, out_hbm, buf_v, acc_v):
        wid = lax.axis_index("s") * nc + lax.axis_index("c")
        n = wid // cpb
        r = wid % cpb
        pltpu.sync_copy(x_hbm.at[n, pl.ds(r * rpw, rpw)], buf_v)
        for cc in range(_C // 16):
            sl = pl.ds(cc * 16, 16)

            def body(i, accs):
                return tuple(a + buf_v[8 * i + j, sl]
                             for j, a in enumerate(accs))

            accs = lax.fori_loop(
                0, rpw // 8, body,
                tuple(jnp.zeros((16,), jnp.float32) for _ in range(8)),
                unroll=2)
            acc = accs[0]
            for a in accs[1:]:
                acc = acc + a
            acc_v[sl] = acc
        pltpu.sync_copy(acc_v, out_hbm.at[wid])

    return k(x)


def _epilogue_kernel(p_ref, wt_ref, b_ref, o_ref):
    m = jnp.sum(p_ref[...], axis=0, keepdims=True) * (1.0 / _HW)
    pooled = jnp.dot(m, wt_ref[...],
                     preferred_element_type=jnp.float32) + b_ref[...]
    o_ref[0] = jnp.broadcast_to(pooled, o_ref.shape[1:])


def kernel(features, depth, W, b):
    del depth
    N, C, H, Wd = features.shape
    O = W.shape[0]
    HW = H * Wd
    x = features.transpose(0, 2, 3, 1).reshape(N, HW, C)
    partials = _sc_partial_sums(x)
    nw = partials.shape[0]
    cpb = nw // N
    wt = W.reshape(O, C).T
    b2 = b.reshape(1, O)
    out = pl.pallas_call(
        _epilogue_kernel,
        grid=(N,),
        in_specs=[
            pl.BlockSpec((cpb, C), lambda i: (i, 0)),
            pl.BlockSpec((C, O), lambda i: (0, 0)),
            pl.BlockSpec((1, O), lambda i: (0, 0)),
        ],
        out_specs=pl.BlockSpec((1, HW, O), lambda i: (i, 0, 0)),
        out_shape=jax.ShapeDtypeStruct((N, HW, O), jnp.float32),
    )(partials, wt, b2)
    return out.reshape(N, H, Wd, O).transpose(0, 3, 1, 2)


# speedup vs baseline: 4.8398x; 3.3155x over previous
import functools

import jax
import jax.numpy as jnp
from jax.experimental import pallas as pl
from jax.experimental.pallas import tpu as pltpu

_S = 2
_K = 2


def _pool_conv_broadcast_kernel(*refs):
    xs = refs[:_S]
    wt_ref, b_ref, o_ref, acc_ref = refs[_S], refs[_S + 1], refs[_S + 2], refs[_S + 3]
    k = pl.program_id(1)
    hw = o_ref.shape[1]
    m = xs[0][0, 0, 0].sum(axis=0, keepdims=True)
    for x in xs[1:]:
        m = m + x[0, 0, 0].sum(axis=0, keepdims=True)

    @pl.when(k == 0)
    def _first():
        acc_ref[...] = m

    @pl.when(k == _K - 1)
    def _emit():
        tot = acc_ref[...] + m
        pooled = jnp.dot(tot * (1.0 / hw), wt_ref[...],
                         preferred_element_type=jnp.float32) + b_ref[...]
        o_ref[0] = jnp.broadcast_to(pooled, o_ref.shape[1:])


def kernel(features, depth, W, b):
    del depth
    N, C, H, Wd = features.shape
    O = W.shape[0]
    HW = H * Wd
    R = HW // (_S * _K)
    x = features.transpose(0, 2, 3, 1).reshape(N, _K, _S, R, C)
    wt = W.reshape(O, C).T
    b2 = b.reshape(1, O)
    x_specs = [
        pl.BlockSpec((1, 1, 1, R, C), lambda i, k, s=s: (i, k, s, 0, 0))
        for s in range(_S)
    ]
    out = pl.pallas_call(
        _pool_conv_broadcast_kernel,
        grid=(N, _K),
        in_specs=x_specs + [
            pl.BlockSpec((C, O), lambda i, k: (0, 0)),
            pl.BlockSpec((1, O), lambda i, k: (0, 0)),
        ],
        out_specs=pl.BlockSpec((1, HW, O), lambda i, k: (i, 0, 0)),
        out_shape=jax.ShapeDtypeStruct((N, HW, O), jnp.float32),
        scratch_shapes=[pltpu.VMEM((1, C), jnp.float32)],
    )(*([x] * _S), wt, b2)
    return out.reshape(N, H, Wd, O).transpose(0, 3, 1, 2)


# speedup vs baseline: 6.0655x; 1.2533x over previous
import jax
import jax.numpy as jnp
from jax.experimental import pallas as pl

_S = 8


def _pool_conv_broadcast_kernel(*refs):
    xs = refs[:_S]
    wt_ref, b_ref, o_ref = refs[_S], refs[_S + 1], refs[_S + 2]
    hw = o_ref.shape[1]
    m = xs[0][0, 0].sum(axis=0, keepdims=True)
    for x in xs[1:]:
        m = m + x[0, 0].sum(axis=0, keepdims=True)
    pooled = jnp.dot(m * (1.0 / hw), wt_ref[...],
                     preferred_element_type=jnp.float32) + b_ref[...]
    o_ref[0] = jnp.broadcast_to(pooled, o_ref.shape[1:])


def kernel(features, depth, W, b):
    del depth
    N, C, H, Wd = features.shape
    O = W.shape[0]
    HW = H * Wd
    R = HW // _S
    x = features.transpose(0, 2, 3, 1).reshape(N, _S, R, C)
    wt = W.reshape(O, C).T
    b2 = b.reshape(1, O)
    x_specs = [
        pl.BlockSpec((1, 1, R, C), lambda i, s=s: (i, s, 0, 0)) for s in range(_S)
    ]
    out = pl.pallas_call(
        _pool_conv_broadcast_kernel,
        grid=(N,),
        in_specs=x_specs + [
            pl.BlockSpec((C, O), lambda i: (0, 0)),
            pl.BlockSpec((1, O), lambda i: (0, 0)),
        ],
        out_specs=pl.BlockSpec((1, HW, O), lambda i: (i, 0, 0)),
        out_shape=jax.ShapeDtypeStruct((N, HW, O), jnp.float32),
    )(*([x] * _S), wt, b2)
    return out.reshape(N, H, Wd, O).transpose(0, 3, 1, 2)
